# Initial kernel scaffold; baseline (speedup 1.0000x reference)
import jax
import jax.numpy as jnp
from jax.experimental import pallas as pl


def kernel(x, edge_index, edge_attr, w0, root0, b0, w1, root1, b1, w_final, b_final):
    raise NotImplementedError("write your pallas kernel here")


# speedup vs baseline: 3.3645x; 3.3645x over previous
import functools
import jax
import jax.numpy as jnp
from jax import lax
from jax.experimental import pallas as pl
from jax.experimental.pallas import tpu as pltpu
from jax.experimental.pallas import tpu_sc as plsc

N = 10000
E = 160000
IN = 128
OUT = 64
K = 5
KK = K * K

ER = 1250
EC = 128
NEG_INF = float("-inf")

CG, NH, EQ = 4, 2, 4
NHALF = N // NH
EQN = E // EQ
CHUNK = 800
GCHUNK = 400
GPT = (4 * E) // 32


def _basis_body(a0_ref, a1_ref, src_ref, b4_ref, fidx_ref):
    a0 = a0_ref[...]
    a1 = a1_ref[...]
    src = src_ref[...]
    f0 = a0 * (K - 1)
    f1 = a1 * (K - 1)
    lo0 = jnp.floor(f0)
    lo1 = jnp.floor(f1)
    t0 = f0 - lo0
    t1 = f1 - lo1
    li0 = jnp.clip(lo0.astype(jnp.int32), 0, K - 1)
    li1 = jnp.clip(lo1.astype(jnp.int32), 0, K - 1)
    j = 0
    for b0 in (0, 1):
        for b1 in (0, 1):
            w = (t0 if b0 else 1.0 - t0) * (t1 if b1 else 1.0 - t1)
            i0 = jnp.clip(li0 + b0, 0, K - 1)
            i1 = jnp.clip(li1 + b1, 0, K - 1)
            b4_ref[j] = w
            fidx_ref[j] = src * KK + (i1 * K + i0)
            j += 1


def _basis(a0, a1, src):
    return pl.pallas_call(
        _basis_body,
        out_shape=[
            jax.ShapeDtypeStruct((4, ER, EC), jnp.float32),
            jax.ShapeDtypeStruct((4, ER, EC), jnp.int32),
        ],
    )(a0, a1, src)


def _matmul_body(x_ref, w_ref, o_ref):
    o_ref[...] = jnp.dot(x_ref[...], w_ref[...],
                         preferred_element_type=jnp.float32)


def _features(x, w2):
    cin = x.shape[1]
    nb = 1000
    return pl.pallas_call(
        _matmul_body,
        grid=(N // nb,),
        in_specs=[
            pl.BlockSpec((nb, cin), lambda i: (i, 0)),
            pl.BlockSpec((cin, KK * OUT), lambda i: (0, 0)),
        ],
        out_specs=pl.BlockSpec((nb, KK * OUT), lambda i: (i, 0)),
        out_shape=jax.ShapeDtypeStruct((N, KK * OUT), jnp.float32),
    )(x, w2)


def _gather_body(y_hbm, fidx_hbm, out_hbm, idx_v, rows_v, sem):
    c = lax.axis_index("c")
    s = lax.axis_index("s")
    wid = s * 2 + c
    base = wid * GPT

    def step(k, _):
        o = base + k * GCHUNK
        pltpu.sync_copy(fidx_hbm.at[pl.ds(o, GCHUNK)], idx_v)
        pltpu.async_copy(y_hbm.at[idx_v], rows_v, sem).wait()
        pltpu.sync_copy(rows_v, out_hbm.at[pl.ds(o, GCHUNK)])
        return 0

    lax.fori_loop(0, GPT // GCHUNK, step, 0)


def _gather(y, fidx):
    mesh = plsc.VectorSubcoreMesh(core_axis_name="c", subcore_axis_name="s")
    f = pl.kernel(
        _gather_body,
        out_type=jax.ShapeDtypeStruct((4 * E, OUT), jnp.float32),
        mesh=mesh,
        compiler_params=pltpu.CompilerParams(use_tc_tiling_on_sc=False),
        scratch_types=[
            pltpu.VMEM((GCHUNK,), jnp.int32),
            pltpu.VMEM((GCHUNK, OUT), jnp.float32),
            pltpu.SemaphoreType.DMA,
        ],
    )
    return f(y, fidx)


def _combine_body(rows_ref, b4_ref, m_ref):
    acc = b4_ref[0] * rows_ref[0]
    for j in range(1, 4):
        acc = acc + b4_ref[j] * rows_ref[j]
    m_ref[...] = acc


def _combine(rows, b4):
    nblk = 100
    eb = E // nblk
    return pl.pallas_call(
        _combine_body,
        grid=(nblk,),
        in_specs=[
            pl.BlockSpec((4, eb, OUT), lambda i: (0, i, 0)),
            pl.BlockSpec((4, eb, 1), lambda i: (0, i, 0)),
        ],
        out_specs=pl.BlockSpec((eb, OUT), lambda i: (i, 0)),
        out_shape=jax.ShapeDtypeStruct((E, OUT), jnp.float32),
    )(rows, b4)


def _scatter_body(m_hbm, dst_hbm, out_hbm, acc, dst_v, m_v):
    c = lax.axis_index("c")
    s = lax.axis_index("s")
    wid = s * 2 + c
    eq = wid & 3
    nh = (wid >> 2) & 1
    cg = wid >> 3
    ebase = eq * EQN
    nbase = nh * NHALF
    cbase = cg * 16

    def init(r, _):
        acc[r] = jnp.full((16,), NEG_INF, jnp.float32)
        return 0

    lax.fori_loop(0, NHALF, init, 0)

    def chunk(k, _):
        e0 = ebase + k * CHUNK
        pltpu.sync_copy(dst_hbm.at[pl.ds(e0 // 16, CHUNK // 16)], dst_v)
        pltpu.sync_copy(m_hbm.at[pl.ds(e0, CHUNK), pl.ds(cbase, 16)], m_v)

        def group(g, _):
            rvec = dst_v[g] - nbase
            for j in range(16):
                r = rvec[j]

                @pl.when((r >= 0) & (r < NHALF))
                def _():
                    row = g * 16 + j
                    acc[r] = jnp.maximum(acc[r], m_v[row])

            return 0

        lax.fori_loop(0, CHUNK // 16, group, 0)
        return 0

    lax.fori_loop(0, EQN // CHUNK, chunk, 0)
    pltpu.sync_copy(acc, out_hbm.at[wid])


def _scatter_max(m, dst16):
    mesh = plsc.VectorSubcoreMesh(core_axis_name="c", subcore_axis_name="s")
    f = pl.kernel(
        _scatter_body,
        out_type=jax.ShapeDtypeStruct((32, NHALF, 16), jnp.float32),
        mesh=mesh,
        compiler_params=pltpu.CompilerParams(use_tc_tiling_on_sc=False),
        scratch_types=[
            pltpu.VMEM((NHALF, 16), jnp.float32),
            pltpu.VMEM((CHUNK // 16, 16), jnp.int32),
            pltpu.VMEM((CHUNK, 16), jnp.float32),
        ],
    )
    return f(m, dst16)


def _merge_body(p_ref, x_ref, root_ref, b_ref, h_ref):
    nb = x_ref.shape[0]
    p = p_ref[...]
    mx = jnp.max(p, axis=2)
    cols = [mx[g, 0] for g in range(4)]
    agg = jnp.concatenate(cols, axis=-1)
    agg = jnp.where(jnp.isfinite(agg), agg, 0.0)
    out = agg + jnp.dot(x_ref[...], root_ref[...],
                        preferred_element_type=jnp.float32) + b_ref[...]
    h_ref[...] = jnp.maximum(out, 0.0)


def _merge(partials, x, root, bias):
    cin = x.shape[1]
    nb = 1000
    nblk = NHALF // nb
    return pl.pallas_call(
        _merge_body,
        grid=(NH, nblk),
        in_specs=[
            pl.BlockSpec((CG, 1, EQ, nb, 16), lambda h, i: (0, h, 0, i, 0)),
            pl.BlockSpec((nb, cin), lambda h, i: (h * nblk + i, 0)),
            pl.BlockSpec((cin, OUT), lambda h, i: (0, 0)),
            pl.BlockSpec((1, OUT), lambda h, i: (0, 0)),
        ],
        out_specs=pl.BlockSpec((nb, OUT), lambda h, i: (h * nblk + i, 0)),
        out_shape=jax.ShapeDtypeStruct((N, OUT), jnp.float32),
    )(partials, x, root, bias)


def _final_body(x_ref, h0_ref, h1_ref, w_ref, b_ref, o_ref):
    w = w_ref[...]
    out = jnp.dot(x_ref[...], w[:IN], preferred_element_type=jnp.float32)
    out = out + jnp.dot(h0_ref[...], w[IN:IN + OUT],
                        preferred_element_type=jnp.float32)
    out = out + jnp.dot(h1_ref[...], w[IN + OUT:],
                        preferred_element_type=jnp.float32)
    o_ref[...] = out + b_ref[...]


def _final(x, h0, h1, wf, bf):
    nb = 1000
    return pl.pallas_call(
        _final_body,
        grid=(N // nb,),
        in_specs=[
            pl.BlockSpec((nb, IN), lambda i: (i, 0)),
            pl.BlockSpec((nb, OUT), lambda i: (i, 0)),
            pl.BlockSpec((nb, OUT), lambda i: (i, 0)),
            pl.BlockSpec((IN + 2 * OUT, OUT), lambda i: (0, 0)),
            pl.BlockSpec((1, OUT), lambda i: (0, 0)),
        ],
        out_specs=pl.BlockSpec((nb, OUT), lambda i: (i, 0)),
        out_shape=jax.ShapeDtypeStruct((N, OUT), jnp.float32),
    )(x, h0, h1, wf, bf)


def _layer(x_in, w2, root, bias, fidx_flat, b4, dst16):
    y = _features(x_in, w2)
    y = y.reshape(KK * N, OUT)
    rows = _gather(y, fidx_flat)
    m = _combine(rows.reshape(4, E, OUT),
                 b4.reshape(4, E, 1))
    partials = _scatter_max(m, dst16)
    partials = partials.reshape(CG, NH, EQ, NHALF, 16)
    return _merge(partials, x_in, root, bias.reshape(1, OUT))


@jax.jit
def kernel(x, edge_index, edge_attr, w0, root0, b0, w1, root1, b1,
           w_final, b_final):
    src = edge_index[0].astype(jnp.int32)
    dst16 = edge_index[1].astype(jnp.int32).reshape(E // 16, 16)
    a0 = edge_attr[:, 0].reshape(ER, EC)
    a1 = edge_attr[:, 1].reshape(ER, EC)
    b4, fidx = _basis(a0, a1, src.reshape(ER, EC))
    fidx_flat = fidx.reshape(4 * E)

    w2_0 = w0.transpose(1, 0, 2).reshape(IN, KK * OUT)
    w2_1 = w1.transpose(1, 0, 2).reshape(OUT, KK * OUT)

    h0 = _layer(x, w2_0, root0, b0, fidx_flat, b4, dst16)
    h1 = _layer(h0, w2_1, root1, b1, fidx_flat, b4, dst16)
    return _final(x, h0, h1, w_final, b_final.reshape(1, OUT))


# speedup vs baseline: 3.4145x; 1.0148x over previous
import functools
import jax
import jax.numpy as jnp
from jax import lax
from jax.experimental import pallas as pl
from jax.experimental.pallas import tpu as pltpu
from jax.experimental.pallas import tpu_sc as plsc

N = 10000
E = 160000
IN = 128
OUT = 64
K = 5
KK = K * K

ER = 1250
EC = 128
NEG_INF = float("-inf")

CG, NH, EQ = 4, 2, 4
NHALF = N // NH
EQN = E // EQ
CHUNK = 800
GCHUNK = 400
GPT = (4 * E) // 32


def _basis_body(a0_ref, a1_ref, src_ref, b4_ref, fidx_ref):
    a0 = a0_ref[...]
    a1 = a1_ref[...]
    src = src_ref[...]
    f0 = a0 * (K - 1)
    f1 = a1 * (K - 1)
    lo0 = jnp.floor(f0)
    lo1 = jnp.floor(f1)
    t0 = f0 - lo0
    t1 = f1 - lo1
    li0 = jnp.clip(lo0.astype(jnp.int32), 0, K - 1)
    li1 = jnp.clip(lo1.astype(jnp.int32), 0, K - 1)
    j = 0
    for b0 in (0, 1):
        for b1 in (0, 1):
            w = (t0 if b0 else 1.0 - t0) * (t1 if b1 else 1.0 - t1)
            i0 = jnp.clip(li0 + b0, 0, K - 1)
            i1 = jnp.clip(li1 + b1, 0, K - 1)
            b4_ref[j] = w
            fidx_ref[j] = src * KK + (i1 * K + i0)
            j += 1


def _basis(a0, a1, src):
    return pl.pallas_call(
        _basis_body,
        out_shape=[
            jax.ShapeDtypeStruct((4, ER, EC), jnp.float32),
            jax.ShapeDtypeStruct((4, ER, EC), jnp.int32),
        ],
    )(a0, a1, src)


def _matmul_body(x_ref, w_ref, o_ref):
    o_ref[...] = jnp.dot(x_ref[...], w_ref[...],
                         preferred_element_type=jnp.float32)


def _features(x, w2):
    cin = x.shape[1]
    nb = 1000
    return pl.pallas_call(
        _matmul_body,
        grid=(N // nb,),
        in_specs=[
            pl.BlockSpec((nb, cin), lambda i: (i, 0)),
            pl.BlockSpec((cin, KK * OUT), lambda i: (0, 0)),
        ],
        out_specs=pl.BlockSpec((nb, KK * OUT), lambda i: (i, 0)),
        out_shape=jax.ShapeDtypeStruct((N, KK * OUT), jnp.float32),
    )(x, w2)


def _gather_body(y_hbm, fidx_hbm, out_hbm,
                 idx_v0, idx_v1, rows_v0, rows_v1, sem0, sem1):
    c = lax.axis_index("c")
    s = lax.axis_index("s")
    wid = s * 2 + c
    base = wid * GPT

    def pair(k2, _):
        oa = base + (2 * k2) * GCHUNK
        ob = oa + GCHUNK
        pltpu.sync_copy(fidx_hbm.at[pl.ds(oa, GCHUNK)], idx_v0)
        cpa = pltpu.async_copy(y_hbm.at[idx_v0], rows_v0, sem0)
        pltpu.sync_copy(fidx_hbm.at[pl.ds(ob, GCHUNK)], idx_v1)
        cpb = pltpu.async_copy(y_hbm.at[idx_v1], rows_v1, sem1)
        cpa.wait()
        pltpu.sync_copy(rows_v0, out_hbm.at[pl.ds(oa, GCHUNK)])
        cpb.wait()
        pltpu.sync_copy(rows_v1, out_hbm.at[pl.ds(ob, GCHUNK)])
        return 0

    lax.fori_loop(0, GPT // (2 * GCHUNK), pair, 0)


def _gather(y, fidx):
    mesh = plsc.VectorSubcoreMesh(core_axis_name="c", subcore_axis_name="s")
    f = pl.kernel(
        _gather_body,
        out_type=jax.ShapeDtypeStruct((4 * E, OUT), jnp.float32),
        mesh=mesh,
        compiler_params=pltpu.CompilerParams(use_tc_tiling_on_sc=False),
        scratch_types=[
            pltpu.VMEM((GCHUNK,), jnp.int32),
            pltpu.VMEM((GCHUNK,), jnp.int32),
            pltpu.VMEM((GCHUNK, OUT), jnp.float32),
            pltpu.VMEM((GCHUNK, OUT), jnp.float32),
            pltpu.SemaphoreType.DMA,
            pltpu.SemaphoreType.DMA,
        ],
    )
    return f(y, fidx)


def _combine_body(rows_ref, b4_ref, m_ref):
    acc = b4_ref[0] * rows_ref[0]
    for j in range(1, 4):
        acc = acc + b4_ref[j] * rows_ref[j]
    m_ref[...] = acc


def _combine(rows, b4):
    nblk = 100
    eb = E // nblk
    return pl.pallas_call(
        _combine_body,
        grid=(nblk,),
        in_specs=[
            pl.BlockSpec((4, eb, OUT), lambda i: (0, i, 0)),
            pl.BlockSpec((4, eb, 1), lambda i: (0, i, 0)),
        ],
        out_specs=pl.BlockSpec((eb, OUT), lambda i: (i, 0)),
        out_shape=jax.ShapeDtypeStruct((E, OUT), jnp.float32),
    )(rows, b4)


def _scatter_body(m_hbm, dst_hbm, out_hbm, acc, dst_v, m_v):
    c = lax.axis_index("c")
    s = lax.axis_index("s")
    wid = s * 2 + c
    eq = wid & 3
    nh = (wid >> 2) & 1
    cg = wid >> 3
    ebase = eq * EQN
    nbase = nh * NHALF
    cbase = cg * 16

    def init(r, _):
        acc[r] = jnp.full((16,), NEG_INF, jnp.float32)
        return 0

    lax.fori_loop(0, NHALF, init, 0)

    def chunk(k, _):
        e0 = ebase + k * CHUNK
        pltpu.sync_copy(dst_hbm.at[pl.ds(e0 // 16, CHUNK // 16)], dst_v)
        pltpu.sync_copy(m_hbm.at[pl.ds(e0, CHUNK), pl.ds(cbase, 16)], m_v)

        def group(g, _):
            rvec = dst_v[g] - nbase
            ok = (rvec >= 0) & (rvec < NHALF)
            rvec = jnp.where(ok, rvec, NHALF)
            for j in range(16):
                r = rvec[j]
                row = g * 16 + j
                acc[r] = jnp.maximum(acc[r], m_v[row])
            return 0

        lax.fori_loop(0, CHUNK // 16, group, 0)
        return 0

    lax.fori_loop(0, EQN // CHUNK, chunk, 0)
    pltpu.sync_copy(acc.at[pl.ds(0, NHALF)], out_hbm.at[wid])


def _scatter_max(m, dst16):
    mesh = plsc.VectorSubcoreMesh(core_axis_name="c", subcore_axis_name="s")
    f = pl.kernel(
        _scatter_body,
        out_type=jax.ShapeDtypeStruct((32, NHALF, 16), jnp.float32),
        mesh=mesh,
        compiler_params=pltpu.CompilerParams(use_tc_tiling_on_sc=False),
        scratch_types=[
            pltpu.VMEM((NHALF + 1, 16), jnp.float32),
            pltpu.VMEM((CHUNK // 16, 16), jnp.int32),
            pltpu.VMEM((CHUNK, 16), jnp.float32),
        ],
    )
    return f(m, dst16)


def _merge_body(p_ref, x_ref, root_ref, b_ref, h_ref):
    nb = x_ref.shape[0]
    p = p_ref[...]
    mx = jnp.max(p, axis=2)
    cols = [mx[g, 0] for g in range(4)]
    agg = jnp.concatenate(cols, axis=-1)
    agg = jnp.where(jnp.isfinite(agg), agg, 0.0)
    out = agg + jnp.dot(x_ref[...], root_ref[...],
                        preferred_element_type=jnp.float32) + b_ref[...]
    h_ref[...] = jnp.maximum(out, 0.0)


def _merge(partials, x, root, bias):
    cin = x.shape[1]
    nb = 1000
    nblk = NHALF // nb
    return pl.pallas_call(
        _merge_body,
        grid=(NH, nblk),
        in_specs=[
            pl.BlockSpec((CG, 1, EQ, nb, 16), lambda h, i: (0, h, 0, i, 0)),
            pl.BlockSpec((nb, cin), lambda h, i: (h * nblk + i, 0)),
            pl.BlockSpec((cin, OUT), lambda h, i: (0, 0)),
            pl.BlockSpec((1, OUT), lambda h, i: (0, 0)),
        ],
        out_specs=pl.BlockSpec((nb, OUT), lambda h, i: (h * nblk + i, 0)),
        out_shape=jax.ShapeDtypeStruct((N, OUT), jnp.float32),
    )(partials, x, root, bias)


def _final_body(x_ref, h0_ref, h1_ref, w_ref, b_ref, o_ref):
    w = w_ref[...]
    out = jnp.dot(x_ref[...], w[:IN], preferred_element_type=jnp.float32)
    out = out + jnp.dot(h0_ref[...], w[IN:IN + OUT],
                        preferred_element_type=jnp.float32)
    out = out + jnp.dot(h1_ref[...], w[IN + OUT:],
                        preferred_element_type=jnp.float32)
    o_ref[...] = out + b_ref[...]


def _final(x, h0, h1, wf, bf):
    nb = 1000
    return pl.pallas_call(
        _final_body,
        grid=(N // nb,),
        in_specs=[
            pl.BlockSpec((nb, IN), lambda i: (i, 0)),
            pl.BlockSpec((nb, OUT), lambda i: (i, 0)),
            pl.BlockSpec((nb, OUT), lambda i: (i, 0)),
            pl.BlockSpec((IN + 2 * OUT, OUT), lambda i: (0, 0)),
            pl.BlockSpec((1, OUT), lambda i: (0, 0)),
        ],
        out_specs=pl.BlockSpec((nb, OUT), lambda i: (i, 0)),
        out_shape=jax.ShapeDtypeStruct((N, OUT), jnp.float32),
    )(x, h0, h1, wf, bf)


def _layer(x_in, w2, root, bias, fidx_flat, b4, dst16):
    y = _features(x_in, w2)
    y = y.reshape(KK * N, OUT)
    rows = _gather(y, fidx_flat)
    m = _combine(rows.reshape(4, E, OUT),
                 b4.reshape(4, E, 1))
    partials = _scatter_max(m, dst16)
    partials = partials.reshape(CG, NH, EQ, NHALF, 16)
    return _merge(partials, x_in, root, bias.reshape(1, OUT))


@jax.jit
def kernel(x, edge_index, edge_attr, w0, root0, b0, w1, root1, b1,
           w_final, b_final):
    src = edge_index[0].astype(jnp.int32)
    dst16 = edge_index[1].astype(jnp.int32).reshape(E // 16, 16)
    a0 = edge_attr[:, 0].reshape(ER, EC)
    a1 = edge_attr[:, 1].reshape(ER, EC)
    b4, fidx = _basis(a0, a1, src.reshape(ER, EC))
    fidx_flat = fidx.reshape(4 * E)

    w2_0 = w0.transpose(1, 0, 2).reshape(IN, KK * OUT)
    w2_1 = w1.transpose(1, 0, 2).reshape(OUT, KK * OUT)

    h0 = _layer(x, w2_0, root0, b0, fidx_flat, b4, dst16)
    h1 = _layer(h0, w2_1, root1, b1, fidx_flat, b4, dst16)
    return _final(x, h0, h1, w_final, b_final.reshape(1, OUT))


# speedup vs baseline: 3.6267x; 1.0622x over previous
import functools
import jax
import jax.numpy as jnp
from jax import lax
from jax.experimental import pallas as pl
from jax.experimental.pallas import tpu as pltpu
from jax.experimental.pallas import tpu_sc as plsc

N = 10000
E = 160000
IN = 128
OUT = 64
K = 5
KK = K * K

ER = 1250
EC = 128
NEG_INF = float("-inf")

CG, NH, EQ = 4, 2, 4
NHALF = N // NH
EQN = E // EQ
CHUNK = 800
GCHUNK = 400
GPT = (4 * E) // 32


def _basis_body(a0_ref, a1_ref, src_ref, b4_ref, fidx_ref):
    a0 = a0_ref[...]
    a1 = a1_ref[...]
    src = src_ref[...]
    f0 = a0 * (K - 1)
    f1 = a1 * (K - 1)
    lo0 = jnp.floor(f0)
    lo1 = jnp.floor(f1)
    t0 = f0 - lo0
    t1 = f1 - lo1
    li0 = jnp.clip(lo0.astype(jnp.int32), 0, K - 1)
    li1 = jnp.clip(lo1.astype(jnp.int32), 0, K - 1)
    j = 0
    for b0 in (0, 1):
        for b1 in (0, 1):
            w = (t0 if b0 else 1.0 - t0) * (t1 if b1 else 1.0 - t1)
            i0 = jnp.clip(li0 + b0, 0, K - 1)
            i1 = jnp.clip(li1 + b1, 0, K - 1)
            b4_ref[j] = w
            fidx_ref[j] = src * KK + (i1 * K + i0)
            j += 1


def _basis(a0, a1, src):
    return pl.pallas_call(
        _basis_body,
        out_shape=[
            jax.ShapeDtypeStruct((4, ER, EC), jnp.float32),
            jax.ShapeDtypeStruct((4, ER, EC), jnp.int32),
        ],
    )(a0, a1, src)


def _matmul_body(x_ref, w_ref, o_ref):
    o_ref[...] = jnp.dot(x_ref[...], w_ref[...],
                         preferred_element_type=jnp.float32)


def _features(x, w2):
    cin = x.shape[1]
    nb = 1000
    return pl.pallas_call(
        _matmul_body,
        grid=(N // nb,),
        in_specs=[
            pl.BlockSpec((nb, cin), lambda i: (i, 0)),
            pl.BlockSpec((cin, KK * OUT), lambda i: (0, 0)),
        ],
        out_specs=pl.BlockSpec((nb, KK * OUT), lambda i: (i, 0)),
        out_shape=jax.ShapeDtypeStruct((N, KK * OUT), jnp.float32),
    )(x, w2)


def _gather_body(y_hbm, fidx_hbm, out_hbm,
                 idx_v0, idx_v1, rows_v0, rows_v1, sem0, sem1):
    c = lax.axis_index("c")
    s = lax.axis_index("s")
    wid = s * 2 + c
    base = wid * GPT

    def pair(k2, _):
        oa = base + (2 * k2) * GCHUNK
        ob = oa + GCHUNK
        pltpu.sync_copy(fidx_hbm.at[pl.ds(oa, GCHUNK)], idx_v0)
        cpa = pltpu.async_copy(y_hbm.at[idx_v0], rows_v0, sem0)
        pltpu.sync_copy(fidx_hbm.at[pl.ds(ob, GCHUNK)], idx_v1)
        cpb = pltpu.async_copy(y_hbm.at[idx_v1], rows_v1, sem1)
        cpa.wait()
        pltpu.sync_copy(rows_v0, out_hbm.at[pl.ds(oa, GCHUNK)])
        cpb.wait()
        pltpu.sync_copy(rows_v1, out_hbm.at[pl.ds(ob, GCHUNK)])
        return 0

    lax.fori_loop(0, GPT // (2 * GCHUNK), pair, 0)


def _gather(y, fidx):
    mesh = plsc.VectorSubcoreMesh(core_axis_name="c", subcore_axis_name="s")
    f = pl.kernel(
        _gather_body,
        out_type=jax.ShapeDtypeStruct((4 * E, OUT), jnp.float32),
        mesh=mesh,
        compiler_params=pltpu.CompilerParams(use_tc_tiling_on_sc=False),
        scratch_types=[
            pltpu.VMEM((GCHUNK,), jnp.int32),
            pltpu.VMEM((GCHUNK,), jnp.int32),
            pltpu.VMEM((GCHUNK, OUT), jnp.float32),
            pltpu.VMEM((GCHUNK, OUT), jnp.float32),
            pltpu.SemaphoreType.DMA,
            pltpu.SemaphoreType.DMA,
        ],
    )
    return f(y, fidx)


def _combine_body(rows_ref, b4_ref, m_ref):
    acc = b4_ref[0] * rows_ref[0]
    for j in range(1, 4):
        acc = acc + b4_ref[j] * rows_ref[j]
    m_ref[...] = acc


def _combine(rows, b4):
    nblk = 100
    eb = E // nblk
    return pl.pallas_call(
        _combine_body,
        grid=(nblk,),
        in_specs=[
            pl.BlockSpec((4, eb, OUT), lambda i: (0, i, 0)),
            pl.BlockSpec((4, eb, 1), lambda i: (0, i, 0)),
        ],
        out_specs=pl.BlockSpec((eb, OUT), lambda i: (i, 0)),
        out_shape=jax.ShapeDtypeStruct((E, OUT), jnp.float32),
    )(rows, b4)


def _scatter_body(m_hbm, dst_hbm, out_hbm, acc, dst_v, m_v):
    c = lax.axis_index("c")
    s = lax.axis_index("s")
    wid = s * 2 + c
    eq = wid & 3
    nh = (wid >> 2) & 1
    cg = wid >> 3
    ebase = eq * EQN
    nbase = nh * NHALF
    cbase = cg * 16

    def init(r, _):
        acc[r] = jnp.full((16,), NEG_INF, jnp.float32)
        return 0

    lax.fori_loop(0, NHALF, init, 0)

    def chunk(k, _):
        e0 = ebase + k * CHUNK
        pltpu.sync_copy(dst_hbm.at[pl.ds(e0 // 16, CHUNK // 16)], dst_v)
        pltpu.sync_copy(m_hbm.at[pl.ds(e0, CHUNK), pl.ds(cbase, 16)], m_v)

        def group(g, _):
            rvec = dst_v[g] - nbase
            ok = (rvec >= 0) & (rvec < NHALF)
            rvec = jnp.where(ok, rvec, NHALF)
            for q in range(4):
                rs = [rvec[4 * q + i] for i in range(4)]
                ms = [m_v[g * 16 + 4 * q + i] for i in range(4)]
                for i in range(4):
                    for j in range(i + 1, 4):
                        eq = rs[j] == rs[i]
                        ms[i] = jnp.where(eq, jnp.maximum(ms[i], ms[j]),
                                          ms[i])
                        rs[j] = jnp.where(eq, NHALF, rs[j])
                vals = [jnp.maximum(acc[rs[i]], ms[i]) for i in range(4)]
                for i in range(4):
                    acc[rs[i]] = vals[i]
            return 0

        lax.fori_loop(0, CHUNK // 16, group, 0)
        return 0

    lax.fori_loop(0, EQN // CHUNK, chunk, 0)
    pltpu.sync_copy(acc.at[pl.ds(0, NHALF)], out_hbm.at[wid])


def _scatter_max(m, dst16):
    mesh = plsc.VectorSubcoreMesh(core_axis_name="c", subcore_axis_name="s")
    f = pl.kernel(
        _scatter_body,
        out_type=jax.ShapeDtypeStruct((32, NHALF, 16), jnp.float32),
        mesh=mesh,
        compiler_params=pltpu.CompilerParams(use_tc_tiling_on_sc=False),
        scratch_types=[
            pltpu.VMEM((NHALF + 1, 16), jnp.float32),
            pltpu.VMEM((CHUNK // 16, 16), jnp.int32),
            pltpu.VMEM((CHUNK, 16), jnp.float32),
        ],
    )
    return f(m, dst16)


def _merge_body(p_ref, x_ref, root_ref, b_ref, h_ref):
    nb = x_ref.shape[0]
    p = p_ref[...]
    mx = jnp.max(p, axis=2)
    cols = [mx[g, 0] for g in range(4)]
    agg = jnp.concatenate(cols, axis=-1)
    agg = jnp.where(jnp.isfinite(agg), agg, 0.0)
    out = agg + jnp.dot(x_ref[...], root_ref[...],
                        preferred_element_type=jnp.float32) + b_ref[...]
    h_ref[...] = jnp.maximum(out, 0.0)


def _merge(partials, x, root, bias):
    cin = x.shape[1]
    nb = 1000
    nblk = NHALF // nb
    return pl.pallas_call(
        _merge_body,
        grid=(NH, nblk),
        in_specs=[
            pl.BlockSpec((CG, 1, EQ, nb, 16), lambda h, i: (0, h, 0, i, 0)),
            pl.BlockSpec((nb, cin), lambda h, i: (h * nblk + i, 0)),
            pl.BlockSpec((cin, OUT), lambda h, i: (0, 0)),
            pl.BlockSpec((1, OUT), lambda h, i: (0, 0)),
        ],
        out_specs=pl.BlockSpec((nb, OUT), lambda h, i: (h * nblk + i, 0)),
        out_shape=jax.ShapeDtypeStruct((N, OUT), jnp.float32),
    )(partials, x, root, bias)


def _final_body(x_ref, h0_ref, h1_ref, w_ref, b_ref, o_ref):
    w = w_ref[...]
    out = jnp.dot(x_ref[...], w[:IN], preferred_element_type=jnp.float32)
    out = out + jnp.dot(h0_ref[...], w[IN:IN + OUT],
                        preferred_element_type=jnp.float32)
    out = out + jnp.dot(h1_ref[...], w[IN + OUT:],
                        preferred_element_type=jnp.float32)
    o_ref[...] = out + b_ref[...]


def _final(x, h0, h1, wf, bf):
    nb = 1000
    return pl.pallas_call(
        _final_body,
        grid=(N // nb,),
        in_specs=[
            pl.BlockSpec((nb, IN), lambda i: (i, 0)),
            pl.BlockSpec((nb, OUT), lambda i: (i, 0)),
            pl.BlockSpec((nb, OUT), lambda i: (i, 0)),
            pl.BlockSpec((IN + 2 * OUT, OUT), lambda i: (0, 0)),
            pl.BlockSpec((1, OUT), lambda i: (0, 0)),
        ],
        out_specs=pl.BlockSpec((nb, OUT), lambda i: (i, 0)),
        out_shape=jax.ShapeDtypeStruct((N, OUT), jnp.float32),
    )(x, h0, h1, wf, bf)


def _layer(x_in, w2, root, bias, fidx_flat, b4, dst16):
    y = _features(x_in, w2)
    y = y.reshape(KK * N, OUT)
    rows = _gather(y, fidx_flat)
    m = _combine(rows.reshape(4, E, OUT),
                 b4.reshape(4, E, 1))
    partials = _scatter_max(m, dst16)
    partials = partials.reshape(CG, NH, EQ, NHALF, 16)
    return _merge(partials, x_in, root, bias.reshape(1, OUT))


@jax.jit
def kernel(x, edge_index, edge_attr, w0, root0, b0, w1, root1, b1,
           w_final, b_final):
    src = edge_index[0].astype(jnp.int32)
    dst16 = edge_index[1].astype(jnp.int32).reshape(E // 16, 16)
    a0 = edge_attr[:, 0].reshape(ER, EC)
    a1 = edge_attr[:, 1].reshape(ER, EC)
    b4, fidx = _basis(a0, a1, src.reshape(ER, EC))
    fidx_flat = fidx.reshape(4 * E)

    w2_0 = w0.transpose(1, 0, 2).reshape(IN, KK * OUT)
    w2_1 = w1.transpose(1, 0, 2).reshape(OUT, KK * OUT)

    h0 = _layer(x, w2_0, root0, b0, fidx_flat, b4, dst16)
    h1 = _layer(h0, w2_1, root1, b1, fidx_flat, b4, dst16)
    return _final(x, h0, h1, w_final, b_final.reshape(1, OUT))


# speedup vs baseline: 3.9369x; 1.0855x over previous
import functools
import jax
import jax.numpy as jnp
from jax import lax
from jax.experimental import pallas as pl
from jax.experimental.pallas import tpu as pltpu
from jax.experimental.pallas import tpu_sc as plsc

N = 10000
E = 160000
IN = 128
OUT = 64
K = 5
KK = K * K

ER = 1250
EC = 128
NEG_INF = float("-inf")

CG, NH, EQ = 4, 2, 4
NHALF = N // NH
EQN = E // EQ
CHUNK = 800
GCHUNK = 400
GPT = (4 * E) // 32


def _basis_body(a0_ref, a1_ref, src_ref, b4_ref, fidx_ref):
    a0 = a0_ref[...]
    a1 = a1_ref[...]
    src = src_ref[...]
    f0 = a0 * (K - 1)
    f1 = a1 * (K - 1)
    lo0 = jnp.floor(f0)
    lo1 = jnp.floor(f1)
    t0 = f0 - lo0
    t1 = f1 - lo1
    li0 = jnp.clip(lo0.astype(jnp.int32), 0, K - 1)
    li1 = jnp.clip(lo1.astype(jnp.int32), 0, K - 1)
    j = 0
    for b0 in (0, 1):
        for b1 in (0, 1):
            w = (t0 if b0 else 1.0 - t0) * (t1 if b1 else 1.0 - t1)
            i0 = jnp.clip(li0 + b0, 0, K - 1)
            i1 = jnp.clip(li1 + b1, 0, K - 1)
            b4_ref[j] = w
            fidx_ref[j] = src * KK + (i1 * K + i0)
            j += 1


def _basis(a0, a1, src):
    return pl.pallas_call(
        _basis_body,
        out_shape=[
            jax.ShapeDtypeStruct((4, ER, EC), jnp.float32),
            jax.ShapeDtypeStruct((4, ER, EC), jnp.int32),
        ],
    )(a0, a1, src)


def _matmul_body(x_ref, w_ref, o_ref):
    o_ref[...] = jnp.dot(x_ref[...], w_ref[...],
                         preferred_element_type=jnp.float32)


def _features(x, w2):
    cin = x.shape[1]
    nb = 1000
    return pl.pallas_call(
        _matmul_body,
        grid=(N // nb,),
        in_specs=[
            pl.BlockSpec((nb, cin), lambda i: (i, 0)),
            pl.BlockSpec((cin, KK * OUT), lambda i: (0, 0)),
        ],
        out_specs=pl.BlockSpec((nb, KK * OUT), lambda i: (i, 0)),
        out_shape=jax.ShapeDtypeStruct((N, KK * OUT), jnp.float32),
    )(x, w2)


def _gather_body(y_hbm, fidx_hbm, out_hbm,
                 idx_v0, idx_v1, rows_v0, rows_v1, sem0, sem1):
    c = lax.axis_index("c")
    s = lax.axis_index("s")
    wid = s * 2 + c
    base = wid * GPT

    def pair(k2, _):
        oa = base + (2 * k2) * GCHUNK
        ob = oa + GCHUNK
        pltpu.sync_copy(fidx_hbm.at[pl.ds(oa, GCHUNK)], idx_v0)
        cpa = pltpu.async_copy(y_hbm.at[idx_v0], rows_v0, sem0)
        pltpu.sync_copy(fidx_hbm.at[pl.ds(ob, GCHUNK)], idx_v1)
        cpb = pltpu.async_copy(y_hbm.at[idx_v1], rows_v1, sem1)
        cpa.wait()
        pltpu.sync_copy(rows_v0, out_hbm.at[pl.ds(oa, GCHUNK)])
        cpb.wait()
        pltpu.sync_copy(rows_v1, out_hbm.at[pl.ds(ob, GCHUNK)])
        return 0

    lax.fori_loop(0, GPT // (2 * GCHUNK), pair, 0)


def _gather(y, fidx):
    mesh = plsc.VectorSubcoreMesh(core_axis_name="c", subcore_axis_name="s")
    f = pl.kernel(
        _gather_body,
        out_type=jax.ShapeDtypeStruct((4 * E, OUT), jnp.float32),
        mesh=mesh,
        compiler_params=pltpu.CompilerParams(use_tc_tiling_on_sc=False),
        scratch_types=[
            pltpu.VMEM((GCHUNK,), jnp.int32),
            pltpu.VMEM((GCHUNK,), jnp.int32),
            pltpu.VMEM((GCHUNK, OUT), jnp.float32),
            pltpu.VMEM((GCHUNK, OUT), jnp.float32),
            pltpu.SemaphoreType.DMA,
            pltpu.SemaphoreType.DMA,
        ],
    )
    return f(y, fidx)


def _combine_body(rows_ref, b4_ref, m_ref):
    acc = b4_ref[0] * rows_ref[0]
    for j in range(1, 4):
        acc = acc + b4_ref[j] * rows_ref[j]
    m_ref[...] = acc


def _combine(rows, b4):
    nblk = 100
    eb = E // nblk
    return pl.pallas_call(
        _combine_body,
        grid=(nblk,),
        in_specs=[
            pl.BlockSpec((4, eb, OUT), lambda i: (0, i, 0)),
            pl.BlockSpec((4, eb, 1), lambda i: (0, i, 0)),
        ],
        out_specs=pl.BlockSpec((eb, OUT), lambda i: (i, 0)),
        out_shape=jax.ShapeDtypeStruct((E, OUT), jnp.float32),
    )(rows, b4)


def _scatter_body(m_hbm, dst_hbm, out_hbm, acc,
                  dst_v, m_v, sd0, sd1, sm0, sm1):
    c = lax.axis_index("c")
    s = lax.axis_index("s")
    wid = s * 2 + c
    eqr = wid & 3
    nh = (wid >> 2) & 1
    cg = wid >> 3
    ebase = eqr * EQN
    nbase = nh * NHALF
    cbase = cg * 16
    nck = EQN // CHUNK
    sds = (sd0, sd1)
    sms = (sm0, sm1)

    def init(r, _):
        acc[r] = jnp.full((16,), NEG_INF, jnp.float32)
        return 0

    def dslice(k):
        e0 = ebase + k * CHUNK
        return dst_hbm.at[pl.ds(e0 // 16, CHUNK // 16)]

    def mslice(k):
        e0 = ebase + k * CHUNK
        return m_hbm.at[pl.ds(e0, CHUNK), pl.ds(cbase, 16)]

    def start(k, b):
        pltpu.async_copy(dslice(k), dst_v.at[b], sds[b])
        pltpu.async_copy(mslice(k), m_v.at[b], sms[b])

    def wait(b):
        pltpu.make_async_copy(dslice(0), dst_v.at[b], sds[b]).wait()
        pltpu.make_async_copy(mslice(0), m_v.at[b], sms[b]).wait()

    def process(b):
        dst_vb = dst_v.at[b]
        m_vb = m_v.at[b]

        def group(g, _):
            rvec = dst_vb[g] - nbase
            ok = (rvec >= 0) & (rvec < NHALF)
            rvec = jnp.where(ok, rvec, NHALF)
            for q in range(4):
                rs = [rvec[4 * q + i] for i in range(4)]
                ms = [m_vb[g * 16 + 4 * q + i] for i in range(4)]
                for i in range(4):
                    for j in range(i + 1, 4):
                        same = rs[j] == rs[i]
                        ms[i] = jnp.where(same, jnp.maximum(ms[i], ms[j]),
                                          ms[i])
                        rs[j] = jnp.where(same, NHALF, rs[j])
                vals = [jnp.maximum(acc[rs[i]], ms[i]) for i in range(4)]
                for i in range(4):
                    acc[rs[i]] = vals[i]
            return 0

        lax.fori_loop(0, CHUNK // 16, group, 0)

    start(0, 0)
    lax.fori_loop(0, NHALF, init, 0)

    def pairloop(k2, _):
        for b in range(2):
            k = 2 * k2 + b
            wait(b)

            @pl.when(k + 1 < nck)
            def _():
                start(k + 1, 1 - b)

            process(b)
        return 0

    lax.fori_loop(0, nck // 2, pairloop, 0)
    pltpu.sync_copy(acc.at[pl.ds(0, NHALF)], out_hbm.at[wid])


def _scatter_max(m, dst16):
    mesh = plsc.VectorSubcoreMesh(core_axis_name="c", subcore_axis_name="s")
    f = pl.kernel(
        _scatter_body,
        out_type=jax.ShapeDtypeStruct((32, NHALF, 16), jnp.float32),
        mesh=mesh,
        compiler_params=pltpu.CompilerParams(use_tc_tiling_on_sc=False),
        scratch_types=[
            pltpu.VMEM((NHALF + 1, 16), jnp.float32),
            pltpu.VMEM((2, CHUNK // 16, 16), jnp.int32),
            pltpu.VMEM((2, CHUNK, 16), jnp.float32),
            pltpu.SemaphoreType.DMA,
            pltpu.SemaphoreType.DMA,
            pltpu.SemaphoreType.DMA,
            pltpu.SemaphoreType.DMA,
        ],
    )
    return f(m, dst16)


def _merge_body(p_ref, x_ref, root_ref, b_ref, h_ref):
    nb = x_ref.shape[0]
    p = p_ref[...]
    mx = jnp.max(p, axis=2)
    cols = [mx[g, 0] for g in range(4)]
    agg = jnp.concatenate(cols, axis=-1)
    agg = jnp.where(jnp.isfinite(agg), agg, 0.0)
    out = agg + jnp.dot(x_ref[...], root_ref[...],
                        preferred_element_type=jnp.float32) + b_ref[...]
    h_ref[...] = jnp.maximum(out, 0.0)


def _merge(partials, x, root, bias):
    cin = x.shape[1]
    nb = 1000
    nblk = NHALF // nb
    return pl.pallas_call(
        _merge_body,
        grid=(NH, nblk),
        in_specs=[
            pl.BlockSpec((CG, 1, EQ, nb, 16), lambda h, i: (0, h, 0, i, 0)),
            pl.BlockSpec((nb, cin), lambda h, i: (h * nblk + i, 0)),
            pl.BlockSpec((cin, OUT), lambda h, i: (0, 0)),
            pl.BlockSpec((1, OUT), lambda h, i: (0, 0)),
        ],
        out_specs=pl.BlockSpec((nb, OUT), lambda h, i: (h * nblk + i, 0)),
        out_shape=jax.ShapeDtypeStruct((N, OUT), jnp.float32),
    )(partials, x, root, bias)


def _final_body(x_ref, h0_ref, h1_ref, w_ref, b_ref, o_ref):
    w = w_ref[...]
    out = jnp.dot(x_ref[...], w[:IN], preferred_element_type=jnp.float32)
    out = out + jnp.dot(h0_ref[...], w[IN:IN + OUT],
                        preferred_element_type=jnp.float32)
    out = out + jnp.dot(h1_ref[...], w[IN + OUT:],
                        preferred_element_type=jnp.float32)
    o_ref[...] = out + b_ref[...]


def _final(x, h0, h1, wf, bf):
    nb = 1000
    return pl.pallas_call(
        _final_body,
        grid=(N // nb,),
        in_specs=[
            pl.BlockSpec((nb, IN), lambda i: (i, 0)),
            pl.BlockSpec((nb, OUT), lambda i: (i, 0)),
            pl.BlockSpec((nb, OUT), lambda i: (i, 0)),
            pl.BlockSpec((IN + 2 * OUT, OUT), lambda i: (0, 0)),
            pl.BlockSpec((1, OUT), lambda i: (0, 0)),
        ],
        out_specs=pl.BlockSpec((nb, OUT), lambda i: (i, 0)),
        out_shape=jax.ShapeDtypeStruct((N, OUT), jnp.float32),
    )(x, h0, h1, wf, bf)


def _layer(x_in, w2, root, bias, fidx_flat, b4, dst16):
    y = _features(x_in, w2)
    y = y.reshape(KK * N, OUT)
    rows = _gather(y, fidx_flat)
    m = _combine(rows.reshape(4, E, OUT),
                 b4.reshape(4, E, 1))
    partials = _scatter_max(m, dst16)
    partials = partials.reshape(CG, NH, EQ, NHALF, 16)
    return _merge(partials, x_in, root, bias.reshape(1, OUT))


@jax.jit
def kernel(x, edge_index, edge_attr, w0, root0, b0, w1, root1, b1,
           w_final, b_final):
    src = edge_index[0].astype(jnp.int32)
    dst16 = edge_index[1].astype(jnp.int32).reshape(E // 16, 16)
    a0 = edge_attr[:, 0].reshape(ER, EC)
    a1 = edge_attr[:, 1].reshape(ER, EC)
    b4, fidx = _basis(a0, a1, src.reshape(ER, EC))
    fidx_flat = fidx.reshape(4 * E)

    w2_0 = w0.transpose(1, 0, 2).reshape(IN, KK * OUT)
    w2_1 = w1.transpose(1, 0, 2).reshape(OUT, KK * OUT)

    h0 = _layer(x, w2_0, root0, b0, fidx_flat, b4, dst16)
    h1 = _layer(h0, w2_1, root1, b1, fidx_flat, b4, dst16)
    return _final(x, h0, h1, w_final, b_final.reshape(1, OUT))


# speedup vs baseline: 3.9405x; 1.0009x over previous
import functools
import jax
import jax.numpy as jnp
from jax import lax
from jax.experimental import pallas as pl
from jax.experimental.pallas import tpu as pltpu
from jax.experimental.pallas import tpu_sc as plsc

N = 10000
E = 160000
IN = 128
OUT = 64
K = 5
KK = K * K

ER = 1250
EC = 128
NEG_INF = float("-inf")

CG, NH, EQ = 4, 2, 4
NHALF = N // NH
EQN = E // EQ
CHUNK = 800
GCHUNK = 400
GPT = (4 * E) // 32


def _basis_body(a0_ref, a1_ref, src_ref, b4_ref, fidx_ref):
    a0 = a0_ref[...]
    a1 = a1_ref[...]
    src = src_ref[...]
    f0 = a0 * (K - 1)
    f1 = a1 * (K - 1)
    lo0 = jnp.floor(f0)
    lo1 = jnp.floor(f1)
    t0 = f0 - lo0
    t1 = f1 - lo1
    li0 = jnp.clip(lo0.astype(jnp.int32), 0, K - 1)
    li1 = jnp.clip(lo1.astype(jnp.int32), 0, K - 1)
    j = 0
    for b0 in (0, 1):
        for b1 in (0, 1):
            w = (t0 if b0 else 1.0 - t0) * (t1 if b1 else 1.0 - t1)
            i0 = jnp.clip(li0 + b0, 0, K - 1)
            i1 = jnp.clip(li1 + b1, 0, K - 1)
            b4_ref[j] = w
            fidx_ref[j] = src * KK + (i1 * K + i0)
            j += 1


def _basis(a0, a1, src):
    return pl.pallas_call(
        _basis_body,
        out_shape=[
            jax.ShapeDtypeStruct((4, ER, EC), jnp.float32),
            jax.ShapeDtypeStruct((4, ER, EC), jnp.int32),
        ],
    )(a0, a1, src)


def _matmul_body(x_ref, w_ref, o_ref):
    o_ref[...] = jnp.dot(x_ref[...], w_ref[...],
                         preferred_element_type=jnp.float32)


def _features(x, w2):
    cin = x.shape[1]
    nb = 1000
    return pl.pallas_call(
        _matmul_body,
        grid=(N // nb,),
        in_specs=[
            pl.BlockSpec((nb, cin), lambda i: (i, 0)),
            pl.BlockSpec((cin, KK * OUT), lambda i: (0, 0)),
        ],
        out_specs=pl.BlockSpec((nb, KK * OUT), lambda i: (i, 0)),
        out_shape=jax.ShapeDtypeStruct((N, KK * OUT), jnp.float32),
    )(x, w2)


def _gather_body(y_hbm, fidx_hbm, out_hbm,
                 idx_v0, idx_v1, rows_v0, rows_v1, sem0, sem1):
    c = lax.axis_index("c")
    s = lax.axis_index("s")
    wid = s * 2 + c
    base = wid * GPT

    def pair(k2, _):
        oa = base + (2 * k2) * GCHUNK
        ob = oa + GCHUNK
        pltpu.sync_copy(fidx_hbm.at[pl.ds(oa, GCHUNK)], idx_v0)
        cpa = pltpu.async_copy(y_hbm.at[idx_v0], rows_v0, sem0)
        pltpu.sync_copy(fidx_hbm.at[pl.ds(ob, GCHUNK)], idx_v1)
        cpb = pltpu.async_copy(y_hbm.at[idx_v1], rows_v1, sem1)
        cpa.wait()
        pltpu.sync_copy(rows_v0, out_hbm.at[pl.ds(oa, GCHUNK)])
        cpb.wait()
        pltpu.sync_copy(rows_v1, out_hbm.at[pl.ds(ob, GCHUNK)])
        return 0

    lax.fori_loop(0, GPT // (2 * GCHUNK), pair, 0)


def _gather(y, fidx):
    mesh = plsc.VectorSubcoreMesh(core_axis_name="c", subcore_axis_name="s")
    f = pl.kernel(
        _gather_body,
        out_type=jax.ShapeDtypeStruct((4 * E, OUT), jnp.float32),
        mesh=mesh,
        compiler_params=pltpu.CompilerParams(use_tc_tiling_on_sc=False),
        scratch_types=[
            pltpu.VMEM((GCHUNK,), jnp.int32),
            pltpu.VMEM((GCHUNK,), jnp.int32),
            pltpu.VMEM((GCHUNK, OUT), jnp.float32),
            pltpu.VMEM((GCHUNK, OUT), jnp.float32),
            pltpu.SemaphoreType.DMA,
            pltpu.SemaphoreType.DMA,
        ],
    )
    return f(y, fidx)


def _combine_body(rows_ref, b4_ref, m_ref):
    acc = b4_ref[0] * rows_ref[0]
    for j in range(1, 4):
        acc = acc + b4_ref[j] * rows_ref[j]
    m_ref[...] = acc


def _combine(rows, b4):
    nblk = 100
    eb = E // nblk
    return pl.pallas_call(
        _combine_body,
        grid=(nblk,),
        in_specs=[
            pl.BlockSpec((4, eb, OUT), lambda i: (0, i, 0)),
            pl.BlockSpec((4, eb, 1), lambda i: (0, i, 0)),
        ],
        out_specs=pl.BlockSpec((eb, OUT), lambda i: (i, 0)),
        out_shape=jax.ShapeDtypeStruct((E, OUT), jnp.float32),
    )(rows, b4)


def _scatter_body(m_hbm, dst_hbm, out_hbm, acc,
                  dst_v, m_v, sd0, sd1, sm0, sm1):
    c = lax.axis_index("c")
    s = lax.axis_index("s")
    wid = s * 2 + c
    eqr = wid & 3
    nh = (wid >> 2) & 1
    cg = wid >> 3
    ebase = eqr * EQN
    nbase = nh * NHALF
    cbase = cg * 16
    nck = EQN // CHUNK
    sds = (sd0, sd1)
    sms = (sm0, sm1)

    def init(r, _):
        acc[r] = jnp.full((16,), NEG_INF, jnp.float32)
        return 0

    def dslice(k):
        e0 = ebase + k * CHUNK
        return dst_hbm.at[pl.ds(e0 // 16, CHUNK // 16)]

    def mslice(k):
        e0 = ebase + k * CHUNK
        return m_hbm.at[pl.ds(e0, CHUNK), pl.ds(cbase, 16)]

    def start(k, b):
        pltpu.async_copy(dslice(k), dst_v.at[b], sds[b])
        pltpu.async_copy(mslice(k), m_v.at[b], sms[b])

    def wait(b):
        pltpu.make_async_copy(dslice(0), dst_v.at[b], sds[b]).wait()
        pltpu.make_async_copy(mslice(0), m_v.at[b], sms[b]).wait()

    def process(b):
        dst_vb = dst_v.at[b]
        m_vb = m_v.at[b]

        def group(g, _):
            rvec = dst_vb[g] - nbase
            ok = (rvec >= 0) & (rvec < NHALF)
            rvec = jnp.where(ok, rvec, NHALF)
            for q in range(4):
                rs = [rvec[4 * q + i] for i in range(4)]
                ms = [m_vb[g * 16 + 4 * q + i] for i in range(4)]
                for i in range(4):
                    for j in range(i + 1, 4):
                        same = rs[j] == rs[i]
                        ms[i] = jnp.where(same, jnp.maximum(ms[i], ms[j]),
                                          ms[i])
                        rs[j] = jnp.where(same, NHALF, rs[j])
                vals = [jnp.maximum(acc[rs[i]], ms[i]) for i in range(4)]
                for i in range(4):
                    acc[rs[i]] = vals[i]
            return 0

        lax.fori_loop(0, CHUNK // 16, group, 0)

    start(0, 0)
    lax.fori_loop(0, NHALF, init, 0)

    def pairloop(k2, _):
        for b in range(2):
            k = 2 * k2 + b
            wait(b)

            @pl.when(k + 1 < nck)
            def _():
                start(k + 1, 1 - b)

            process(b)
        return 0

    lax.fori_loop(0, nck // 2, pairloop, 0)
    pltpu.sync_copy(acc.at[pl.ds(0, NHALF)], out_hbm.at[wid])


def _scatter_max(m, dst16):
    mesh = plsc.VectorSubcoreMesh(core_axis_name="c", subcore_axis_name="s")
    f = pl.kernel(
        _scatter_body,
        out_type=jax.ShapeDtypeStruct((32, NHALF, 16), jnp.float32),
        mesh=mesh,
        compiler_params=pltpu.CompilerParams(use_tc_tiling_on_sc=False),
        scratch_types=[
            pltpu.VMEM((NHALF + 1, 16), jnp.float32),
            pltpu.VMEM((2, CHUNK // 16, 16), jnp.int32),
            pltpu.VMEM((2, CHUNK, 16), jnp.float32),
            pltpu.SemaphoreType.DMA,
            pltpu.SemaphoreType.DMA,
            pltpu.SemaphoreType.DMA,
            pltpu.SemaphoreType.DMA,
        ],
    )
    return f(m, dst16)


def _merge_value(p_ref, x_ref, root_ref, b_ref):
    p = p_ref[...]
    mx = jnp.max(p, axis=2)
    cols = [mx[g, 0] for g in range(4)]
    agg = jnp.concatenate(cols, axis=-1)
    agg = jnp.where(jnp.isfinite(agg), agg, 0.0)
    out = agg + jnp.dot(x_ref[...], root_ref[...],
                        preferred_element_type=jnp.float32) + b_ref[...]
    return jnp.maximum(out, 0.0)


def _merge_body(p_ref, x_ref, root_ref, b_ref, h_ref):
    h_ref[...] = _merge_value(p_ref, x_ref, root_ref, b_ref)


def _merge_feat_body(p_ref, x_ref, root_ref, b_ref, w2_ref, h_ref, y_ref):
    h = _merge_value(p_ref, x_ref, root_ref, b_ref)
    h_ref[...] = h
    y_ref[...] = jnp.dot(h, w2_ref[...], preferred_element_type=jnp.float32)


def _merge(partials, x, root, bias):
    cin = x.shape[1]
    nb = 1000
    nblk = NHALF // nb
    return pl.pallas_call(
        _merge_body,
        grid=(NH, nblk),
        in_specs=[
            pl.BlockSpec((CG, 1, EQ, nb, 16), lambda h, i: (0, h, 0, i, 0)),
            pl.BlockSpec((nb, cin), lambda h, i: (h * nblk + i, 0)),
            pl.BlockSpec((cin, OUT), lambda h, i: (0, 0)),
            pl.BlockSpec((1, OUT), lambda h, i: (0, 0)),
        ],
        out_specs=pl.BlockSpec((nb, OUT), lambda h, i: (h * nblk + i, 0)),
        out_shape=jax.ShapeDtypeStruct((N, OUT), jnp.float32),
    )(partials, x, root, bias)


def _merge_feat(partials, x, root, bias, w2):
    cin = x.shape[1]
    nb = 1000
    nblk = NHALF // nb
    return pl.pallas_call(
        _merge_feat_body,
        grid=(NH, nblk),
        in_specs=[
            pl.BlockSpec((CG, 1, EQ, nb, 16), lambda h, i: (0, h, 0, i, 0)),
            pl.BlockSpec((nb, cin), lambda h, i: (h * nblk + i, 0)),
            pl.BlockSpec((cin, OUT), lambda h, i: (0, 0)),
            pl.BlockSpec((1, OUT), lambda h, i: (0, 0)),
            pl.BlockSpec((OUT, KK * OUT), lambda h, i: (0, 0)),
        ],
        out_specs=[
            pl.BlockSpec((nb, OUT), lambda h, i: (h * nblk + i, 0)),
            pl.BlockSpec((nb, KK * OUT), lambda h, i: (h * nblk + i, 0)),
        ],
        out_shape=[
            jax.ShapeDtypeStruct((N, OUT), jnp.float32),
            jax.ShapeDtypeStruct((N, KK * OUT), jnp.float32),
        ],
    )(partials, x, root, bias, w2)


def _final_body(x_ref, h0_ref, h1_ref, w_ref, b_ref, o_ref):
    w = w_ref[...]
    out = jnp.dot(x_ref[...], w[:IN], preferred_element_type=jnp.float32)
    out = out + jnp.dot(h0_ref[...], w[IN:IN + OUT],
                        preferred_element_type=jnp.float32)
    out = out + jnp.dot(h1_ref[...], w[IN + OUT:],
                        preferred_element_type=jnp.float32)
    o_ref[...] = out + b_ref[...]


def _final(x, h0, h1, wf, bf):
    nb = 1000
    return pl.pallas_call(
        _final_body,
        grid=(N // nb,),
        in_specs=[
            pl.BlockSpec((nb, IN), lambda i: (i, 0)),
            pl.BlockSpec((nb, OUT), lambda i: (i, 0)),
            pl.BlockSpec((nb, OUT), lambda i: (i, 0)),
            pl.BlockSpec((IN + 2 * OUT, OUT), lambda i: (0, 0)),
            pl.BlockSpec((1, OUT), lambda i: (0, 0)),
        ],
        out_specs=pl.BlockSpec((nb, OUT), lambda i: (i, 0)),
        out_shape=jax.ShapeDtypeStruct((N, OUT), jnp.float32),
    )(x, h0, h1, wf, bf)


def _sparse_part(y, fidx_flat, b4, dst16):
    rows = _gather(y.reshape(KK * N, OUT), fidx_flat)
    m = _combine(rows.reshape(4, E, OUT),
                 b4.reshape(4, E, 1))
    partials = _scatter_max(m, dst16)
    return partials.reshape(CG, NH, EQ, NHALF, 16)


@jax.jit
def kernel(x, edge_index, edge_attr, w0, root0, b0, w1, root1, b1,
           w_final, b_final):
    src = edge_index[0].astype(jnp.int32)
    dst16 = edge_index[1].astype(jnp.int32).reshape(E // 16, 16)
    a0 = edge_attr[:, 0].reshape(ER, EC)
    a1 = edge_attr[:, 1].reshape(ER, EC)
    b4, fidx = _basis(a0, a1, src.reshape(ER, EC))
    fidx_flat = fidx.reshape(4 * E)

    w2_0 = w0.transpose(1, 0, 2).reshape(IN, KK * OUT)
    w2_1 = w1.transpose(1, 0, 2).reshape(OUT, KK * OUT)

    y0 = _features(x, w2_0)
    p0 = _sparse_part(y0, fidx_flat, b4, dst16)
    h0, y1 = _merge_feat(p0, x, root0, b0.reshape(1, OUT), w2_1)
    p1 = _sparse_part(y1, fidx_flat, b4, dst16)
    h1 = _merge(p1, h0, root1, b1.reshape(1, OUT))
    return _final(x, h0, h1, w_final, b_final.reshape(1, OUT))
